# baseline (speedup 1.0000x reference)
import numpy as np
import jax

jax.config.update("jax_enable_x64", True)
import jax.numpy as jnp
from jax.experimental import pallas as pl

_INPUT_DIM = 64
_NUM_PROJ = 16
_NUM_BINS = 9
_L = 65536
_OUTW = _L // 8

_W4 = np.zeros((4, _NUM_PROJ), np.float32)
for _i in range(_NUM_PROJ):
    _W4[_i // 4, _i] = 10.0 ** (3 - _i % 4)


def _lsh_block(xt_ref, pt_ref, w4_ref, hi_ref, lo_ref):
    xt = xt_ref[:]
    s = jnp.sum(xt * xt, axis=0, keepdims=True)
    nrm = jnp.maximum(jnp.sqrt(s), 1e-12)
    xn = (xt / nrm).astype(jnp.bfloat16)
    h = jnp.dot(pt_ref[:], xn, preferred_element_type=jnp.float32)
    d = jnp.clip(jnp.ceil(h * 4.5), -4.0, 5.0)
    g4 = jnp.dot(w4_ref[:], d, preferred_element_type=jnp.float32) + 4444.0
    hi8 = g4[0:1, :].astype(jnp.int32) * 10000 + g4[1:2, :].astype(jnp.int32)
    lo8 = g4[2:3, :].astype(jnp.int32) * 10000 + g4[3:4, :].astype(jnp.int32)
    hu = hi8.astype(jnp.uint32)
    lu = lo8.astype(jnp.uint32)
    a = hu >> 16
    b = hu & jnp.uint32(0xFFFF)
    m = a * jnp.uint32(57600) + b * jnp.uint32(1525)
    p0 = b * jnp.uint32(57600)
    t1 = p0 + (m << 16)
    c1 = (t1 < p0).astype(jnp.uint32)
    t2 = t1 + lu
    c2 = (t2 < t1).astype(jnp.uint32)
    w_lo = t2.astype(jnp.int32)
    w_hi = (a * jnp.uint32(1525) + (m >> 16) + c1 + c2).astype(jnp.int32)
    hi_ref[:] = jnp.concatenate(
        [w_hi[:, c * _OUTW:(c + 1) * _OUTW] for c in range(8)], axis=0)
    lo_ref[:] = jnp.concatenate(
        [w_lo[:, c * _OUTW:(c + 1) * _OUTW] for c in range(8)], axis=0)


def kernel(x, projection_mat):
    n = x.shape[0]
    nsteps = -(-n // _L)
    xt = jnp.swapaxes(x, 0, 1)
    pt = jnp.swapaxes(projection_mat, 0, 1).astype(jnp.bfloat16)
    hi, lo = pl.pallas_call(
        _lsh_block,
        grid=(nsteps,),
        in_specs=[
            pl.BlockSpec((_INPUT_DIM, _L), lambda i: (i * 0, i)),
            pl.BlockSpec((_NUM_PROJ, _INPUT_DIM), lambda i: (i * 0, i * 0)),
            pl.BlockSpec((4, _NUM_PROJ), lambda i: (i * 0, i * 0)),
        ],
        out_specs=[
            pl.BlockSpec((8, _OUTW), lambda i: (i, i * 0)),
            pl.BlockSpec((8, _OUTW), lambda i: (i, i * 0)),
        ],
        out_shape=[
            jax.ShapeDtypeStruct((8 * nsteps, _OUTW), jnp.int32),
            jax.ShapeDtypeStruct((8 * nsteps, _OUTW), jnp.int32),
        ],
    )(xt, pt, jnp.asarray(_W4))
    wh = hi.reshape(-1)[:n].astype(jnp.uint32).astype(jnp.uint64)
    wl = lo.reshape(-1)[:n].astype(jnp.uint32).astype(jnp.uint64)
    return ((wh << 32) | wl).astype(jnp.int64)


# speedup vs baseline: 1.0669x; 1.0669x over previous
import numpy as np
import jax

jax.config.update("jax_enable_x64", True)
import jax.numpy as jnp
from jax.experimental import pallas as pl

_INPUT_DIM = 64
_NUM_PROJ = 16
_NUM_BINS = 9
_L = 65536
_OUTW = _L // 8

_W4 = np.zeros((4, _NUM_PROJ), np.float32)
for _i in range(_NUM_PROJ):
    _W4[_i // 4, _i] = 10.0 ** (3 - _i % 4)


def _lsh_block(xt_ref, pt_ref, w4_ref, hi_ref, lo_ref):
    xt = xt_ref[:]
    s = jnp.sum(xt * xt, axis=0, keepdims=True)
    nrm = jnp.maximum(jnp.sqrt(s), 1e-12)
    xn = (xt / nrm).astype(jnp.bfloat16)
    h = jnp.dot(pt_ref[:], xn, preferred_element_type=jnp.float32)
    d = jnp.clip(jnp.ceil(h * 4.5), -4.0, 5.0)
    g4 = jnp.dot(w4_ref[:], d, preferred_element_type=jnp.float32) + 4444.0
    hi = g4[0:1, :].astype(jnp.int32) * 10000 + g4[1:2, :].astype(jnp.int32)
    lo = g4[2:3, :].astype(jnp.int32) * 10000 + g4[3:4, :].astype(jnp.int32)
    hi_ref[:] = jnp.concatenate(
        [hi[:, c * _OUTW:(c + 1) * _OUTW] for c in range(8)], axis=0)
    lo_ref[:] = jnp.concatenate(
        [lo[:, c * _OUTW:(c + 1) * _OUTW] for c in range(8)], axis=0)


def kernel(x, projection_mat):
    n = x.shape[0]
    nsteps = -(-n // _L)
    xt = jnp.swapaxes(x, 0, 1)
    pt = jnp.swapaxes(projection_mat, 0, 1).astype(jnp.bfloat16)
    hi, lo = pl.pallas_call(
        _lsh_block,
        grid=(nsteps,),
        in_specs=[
            pl.BlockSpec((_INPUT_DIM, _L), lambda i: (i * 0, i)),
            pl.BlockSpec((_NUM_PROJ, _INPUT_DIM), lambda i: (i * 0, i * 0)),
            pl.BlockSpec((4, _NUM_PROJ), lambda i: (i * 0, i * 0)),
        ],
        out_specs=[
            pl.BlockSpec((8, _OUTW), lambda i: (i, i * 0)),
            pl.BlockSpec((8, _OUTW), lambda i: (i, i * 0)),
        ],
        out_shape=[
            jax.ShapeDtypeStruct((8 * nsteps, _OUTW), jnp.int32),
            jax.ShapeDtypeStruct((8 * nsteps, _OUTW), jnp.int32),
        ],
    )(xt, pt, jnp.asarray(_W4))
    hi64 = hi.reshape(-1)[:n].astype(jnp.int64)
    lo64 = lo.reshape(-1)[:n].astype(jnp.int64)
    return hi64 * (10**8) + lo64
